# baseline (speedup 1.0000x reference)
import functools

import jax
import jax.numpy as jnp
from jax import lax
from jax.experimental import pallas as pl
from jax.experimental.pallas import tpu as pltpu
from jax.experimental.pallas import tpu_sc as plsc

AVG_NEIGH = 32.0


def _up_body(x_ref, w_ref, o_ref):
    o_ref[...] = jnp.dot(x_ref[...], w_ref[...],
                         preferred_element_type=jnp.float32)


def _mod_body(rad_ref, edge_ref, wr1_ref, wr2_ref, wedge_ref, o_ref):
    eb = o_ref.shape[0]
    d_rad = wr1_ref.shape[0]
    d_edge = wedge_ref.shape[0]
    radf = rad_ref[...]
    edgef = edge_ref[...]
    w1 = wr1_ref[...]
    w2 = wr2_ref[...]
    we = wedge_ref[...]
    npr = 128 // d_rad
    rws = []
    for j in range(npr):
        t = jnp.dot(radf[:, d_rad * j:d_rad * (j + 1)], w1,
                    preferred_element_type=jnp.float32)
        t = t * jax.nn.sigmoid(t)
        rws.append(jnp.dot(t, w2, preferred_element_type=jnp.float32))
    rw = jnp.stack(rws, axis=1).reshape(eb, 128)
    npe = 128 // d_edge
    ews = []
    for j in range(npe):
        ews.append(jnp.dot(edgef[:, d_edge * j:d_edge * (j + 1)], we,
                           preferred_element_type=jnp.float32))
    ew = jnp.stack(ews, axis=1).reshape(eb, 128)
    o_ref[...] = rw * ew


def _down_body(a_ref, w_ref, o_ref):
    a = (a_ref[0] + a_ref[1]) * (1.0 / AVG_NEIGH)
    o_ref[...] = jnp.dot(a, w_ref[...], preferred_element_type=jnp.float32)


def _linear_up(node_features, w_up):
    n, d = node_features.shape
    bn = 1000
    return pl.pallas_call(
        _up_body,
        grid=(n // bn,),
        in_specs=[
            pl.BlockSpec((bn, d), lambda i: (i, 0)),
            pl.BlockSpec((d, d), lambda i: (0, 0)),
        ],
        out_specs=pl.BlockSpec((bn, d), lambda i: (i, 0)),
        out_shape=jax.ShapeDtypeStruct((n, d), jnp.float32),
    )(node_features, w_up)


def _edge_mod(radial, edge, w_r1, w_r2, w_edge):
    e_total = radial.shape[0] * radial.shape[1] // w_r1.shape[0]
    d_rad = w_r1.shape[0]
    d_edge = w_edge.shape[0]
    hid = w_r1.shape[1]
    d = w_r2.shape[1]
    be = 6400
    br = be * d_rad // 128
    bt = be * d_edge // 128
    return pl.pallas_call(
        _mod_body,
        grid=(e_total // be,),
        in_specs=[
            pl.BlockSpec((br, 128), lambda i: (i, 0)),
            pl.BlockSpec((bt, 128), lambda i: (i, 0)),
            pl.BlockSpec((d_rad, hid), lambda i: (0, 0)),
            pl.BlockSpec((hid, d), lambda i: (0, 0)),
            pl.BlockSpec((d_edge, d), lambda i: (0, 0)),
        ],
        out_specs=pl.BlockSpec((be, d), lambda i: (i, 0)),
        out_shape=jax.ShapeDtypeStruct((e_total, d), jnp.float32),
    )(radial, edge, w_r1, w_r2, w_edge)


def _linear_down(agg2, w_down):
    _, n, d = agg2.shape
    bn = 1000
    return pl.pallas_call(
        _down_body,
        grid=(n // bn,),
        in_specs=[
            pl.BlockSpec((2, bn, d), lambda i: (0, i, 0)),
            pl.BlockSpec((d, d), lambda i: (0, 0)),
        ],
        out_specs=pl.BlockSpec((bn, d), lambda i: (i, 0)),
        out_shape=jax.ShapeDtypeStruct((n, d), jnp.float32),
    )(agg2, w_down)


_K = 80
_ZROWS = 80
_WROWS = 200


def _sc_scatter(h, mod, senders, receivers):
    n, d = h.shape
    e = senders.shape[0]
    info = plsc.get_sparse_core_info()
    nc, ns = info.num_cores, info.num_subcores
    nw = nc * ns
    e_per_tile = e // nw
    assert e_per_tile * nw == e and e_per_tile % _K == 0
    n_chunks = e_per_tile // _K
    n_zero_blocks = n // _ZROWS
    n_wb_blocks = n // _WROWS
    assert n_zero_blocks * _ZROWS == n and n_wb_blocks * _WROWS == n

    assert n_chunks % 2 == 1 and n_chunks >= 3
    n_pairs_main = (n_chunks - 3) // 2

    mesh = plsc.VectorSubcoreMesh(core_axis_name="c", subcore_axis_name="s",
                                  num_cores=nc, num_subcores=ns)

    @functools.partial(
        pl.kernel,
        mesh=mesh,
        out_type=jax.ShapeDtypeStruct((nc, n, d), jnp.float32),
        scratch_types=[
            pltpu.VMEM((_K,), jnp.int32),
            pltpu.VMEM((_K,), jnp.int32),
            pltpu.VMEM((_K,), jnp.int32),
            pltpu.VMEM((_K,), jnp.int32),
            pltpu.VMEM((_K, d), jnp.float32),
            pltpu.VMEM((_K, d), jnp.float32),
            pltpu.VMEM((_K, d), jnp.float32),
            pltpu.VMEM((_K, d), jnp.float32),
            pltpu.VMEM_SHARED((n, d), jnp.float32),
            pltpu.SemaphoreType.DMA,
            pltpu.SemaphoreType.DMA,
            pltpu.SemaphoreType.DMA,
            pltpu.SemaphoreType.DMA,
            pltpu.SemaphoreType.DMA,
            pltpu.SemaphoreType.DMA,
        ],
    )
    def body(h_hbm, mod_hbm, send_hbm, recv_hbm, out_hbm,
             sidx0, sidx1, ridx0, ridx1, hrows0, hrows1, mrows0, mrows1,
             agg,
             in0, in1, g0, g1, s0, s1):
        cid = lax.axis_index("c")
        sid = lax.axis_index("s")
        wid = cid * ns + sid

        sidx = (sidx0, sidx1)
        ridx = (ridx0, ridx1)
        hrows = (hrows0, hrows1)
        mrows = (mrows0, mrows1)
        insem = (in0, in1)
        gsem = (g0, g1)
        ssem = (s0, s1)

        zero = jnp.zeros((16,), jnp.float32)

        def zfill(i, carry):
            for c in range(d // 16):
                mrows0[i, pl.ds(c * 16, 16)] = zero
            return carry

        lax.fori_loop(0, _ZROWS, zfill, 0)
        for b in range(n_zero_blocks):
            @pl.when(b % ns == sid)
            def _():
                pltpu.sync_copy(mrows0, agg.at[pl.ds(b * _ZROWS, _ZROWS)])
        plsc.subcore_barrier()

        base0 = wid * e_per_tile

        def issue_loads(base, b):
            pltpu.async_copy(send_hbm.at[pl.ds(base, _K)], sidx[b], insem[b])
            pltpu.async_copy(recv_hbm.at[pl.ds(base, _K)], ridx[b], insem[b])
            pltpu.async_copy(mod_hbm.at[pl.ds(base, _K)], mrows[b], insem[b])

        def wait_loads(base, b):
            pltpu.make_async_copy(send_hbm.at[pl.ds(base, _K)], sidx[b],
                                  insem[b]).wait()
            pltpu.make_async_copy(recv_hbm.at[pl.ds(base, _K)], ridx[b],
                                  insem[b]).wait()
            pltpu.make_async_copy(mod_hbm.at[pl.ds(base, _K)], mrows[b],
                                  insem[b]).wait()

        def issue_gather(b):
            pltpu.async_copy(h_hbm.at[sidx[b]], hrows[b], gsem[b])

        def wait_gather(b):
            pltpu.make_async_copy(h_hbm.at[sidx[b]], hrows[b], gsem[b]).wait()

        def mul(b):
            def mrow(i, carry):
                for c in range(d // 16):
                    sl = pl.ds(c * 16, 16)
                    mrows[b][i, sl] = mrows[b][i, sl] * hrows[b][i, sl]
                return carry
            lax.fori_loop(0, _K, mrow, 0)

        def issue_scatter(b):
            pltpu.async_copy(mrows[b], agg.at[ridx[b]], ssem[b], add=True)

        def wait_scatter(b):
            pltpu.make_async_copy(mrows[b], agg.at[ridx[b]], ssem[b]).wait()

        issue_loads(base0, 0)
        issue_loads(base0 + _K, 1)
        wait_loads(base0, 0)
        issue_gather(0)

        def pair(i, carry):
            c0 = base0 + (2 * i) * _K
            wait_gather(0)
            mul(0)
            issue_scatter(0)
            wait_loads(c0 + _K, 1)
            issue_gather(1)
            wait_scatter(0)
            issue_loads(c0 + 2 * _K, 0)
            wait_gather(1)
            mul(1)
            issue_scatter(1)
            wait_loads(c0 + 2 * _K, 0)
            issue_gather(0)
            wait_scatter(1)
            issue_loads(c0 + 3 * _K, 1)
            return carry

        lax.fori_loop(0, n_pairs_main, pair, 0)

        cA = base0 + (n_chunks - 3) * _K
        wait_gather(0)
        mul(0)
        issue_scatter(0)
        wait_loads(cA + _K, 1)
        issue_gather(1)
        wait_scatter(0)
        issue_loads(cA + 2 * _K, 0)
        wait_gather(1)
        mul(1)
        issue_scatter(1)
        wait_loads(cA + 2 * _K, 0)
        issue_gather(0)
        wait_scatter(1)
        wait_gather(0)
        mul(0)
        issue_scatter(0)
        wait_scatter(0)
        plsc.subcore_barrier()

        for b in range(n_wb_blocks):
            @pl.when(b % ns == sid)
            def _():
                pltpu.sync_copy(agg.at[pl.ds(b * _WROWS, _WROWS)],
                                out_hbm.at[cid, pl.ds(b * _WROWS, _WROWS)])

    return body(h, mod, senders, receivers)


def kernel(node_features, edge_features, radial_embedding, senders, receivers,
           W_up, W_r1, W_r2, W_edge, W_down):
    h = _linear_up(node_features, W_up)
    rad_flat = radial_embedding.reshape(-1, 128)
    edge_flat = edge_features.reshape(-1, 128)
    mod = _edge_mod(rad_flat, edge_flat, W_r1, W_r2, W_edge)
    agg2 = _sc_scatter(h, mod, senders, receivers)
    return _linear_down(agg2, W_down)


# speedup vs baseline: 1.4172x; 1.4172x over previous
import functools

import jax
import jax.numpy as jnp
from jax import lax
from jax.experimental import pallas as pl
from jax.experimental.pallas import tpu as pltpu
from jax.experimental.pallas import tpu_sc as plsc

AVG_NEIGH = 32.0


def _up_body(x_ref, w_ref, o_ref):
    o_ref[...] = jnp.dot(x_ref[...], w_ref[...],
                         preferred_element_type=jnp.float32)


def _mod_body(rad_ref, edge_ref, wr1_ref, wr2_ref, wedge_ref, o_ref):
    t = jnp.dot(rad_ref[...], wr1_ref[...],
                preferred_element_type=jnp.float32)
    t = t * jax.nn.sigmoid(t)
    rw = jnp.dot(t, wr2_ref[...], preferred_element_type=jnp.float32)
    ew = jnp.dot(edge_ref[...], wedge_ref[...],
                 preferred_element_type=jnp.float32)
    o_ref[...] = rw * ew


def _down_body(a_ref, w_ref, o_ref):
    a = (a_ref[0] + a_ref[1]) * (1.0 / AVG_NEIGH)
    o_ref[...] = jnp.dot(a, w_ref[...], preferred_element_type=jnp.float32)


def _linear_up(node_features, w_up):
    n, d = node_features.shape
    bn = 1000
    return pl.pallas_call(
        _up_body,
        grid=(n // bn,),
        in_specs=[
            pl.BlockSpec((bn, d), lambda i: (i, 0)),
            pl.BlockSpec((d, d), lambda i: (0, 0)),
        ],
        out_specs=pl.BlockSpec((bn, d), lambda i: (i, 0)),
        out_shape=jax.ShapeDtypeStruct((n, d), jnp.float32),
    )(node_features, w_up)


def _edge_mod(radial, edge, w_r1, w_r2, w_edge):
    e, d_rad = radial.shape
    d_edge = edge.shape[1]
    hid = w_r1.shape[1]
    d = w_r2.shape[1]
    be = 4000
    return pl.pallas_call(
        _mod_body,
        grid=(e // be,),
        in_specs=[
            pl.BlockSpec((be, d_rad), lambda i: (i, 0)),
            pl.BlockSpec((be, d_edge), lambda i: (i, 0)),
            pl.BlockSpec((d_rad, hid), lambda i: (0, 0)),
            pl.BlockSpec((hid, d), lambda i: (0, 0)),
            pl.BlockSpec((d_edge, d), lambda i: (0, 0)),
        ],
        out_specs=pl.BlockSpec((be, d), lambda i: (i, 0)),
        out_shape=jax.ShapeDtypeStruct((e, d), jnp.float32),
    )(radial, edge, w_r1, w_r2, w_edge)


def _linear_down(agg2, w_down):
    _, n, d = agg2.shape
    bn = 1000
    return pl.pallas_call(
        _down_body,
        grid=(n // bn,),
        in_specs=[
            pl.BlockSpec((2, bn, d), lambda i: (0, i, 0)),
            pl.BlockSpec((d, d), lambda i: (0, 0)),
        ],
        out_specs=pl.BlockSpec((bn, d), lambda i: (i, 0)),
        out_shape=jax.ShapeDtypeStruct((n, d), jnp.float32),
    )(agg2, w_down)


_K = 80
_ZROWS = 80
_WROWS = 200


def _sc_scatter(h, mod, senders, receivers):
    n, d = h.shape
    e = senders.shape[0]
    info = plsc.get_sparse_core_info()
    nc, ns = info.num_cores, info.num_subcores
    nw = nc * ns
    e_per_tile = e // nw
    assert e_per_tile * nw == e and e_per_tile % _K == 0
    n_chunks = e_per_tile // _K
    n_zero_blocks = n // _ZROWS
    n_wb_blocks = n // _WROWS
    assert n_zero_blocks * _ZROWS == n and n_wb_blocks * _WROWS == n

    assert n_chunks % 2 == 1 and n_chunks >= 3
    n_pairs_main = (n_chunks - 3) // 2

    mesh = plsc.VectorSubcoreMesh(core_axis_name="c", subcore_axis_name="s",
                                  num_cores=nc, num_subcores=ns)

    @functools.partial(
        pl.kernel,
        mesh=mesh,
        out_type=jax.ShapeDtypeStruct((nc, n, d), jnp.float32),
        scratch_types=[
            pltpu.VMEM((_K,), jnp.int32),
            pltpu.VMEM((_K,), jnp.int32),
            pltpu.VMEM((_K,), jnp.int32),
            pltpu.VMEM((_K,), jnp.int32),
            pltpu.VMEM((_K, d), jnp.float32),
            pltpu.VMEM((_K, d), jnp.float32),
            pltpu.VMEM((_K, d), jnp.float32),
            pltpu.VMEM((_K, d), jnp.float32),
            pltpu.VMEM_SHARED((n, d), jnp.float32),
            pltpu.SemaphoreType.DMA,
            pltpu.SemaphoreType.DMA,
            pltpu.SemaphoreType.DMA,
            pltpu.SemaphoreType.DMA,
            pltpu.SemaphoreType.DMA,
            pltpu.SemaphoreType.DMA,
        ],
    )
    def body(h_hbm, mod_hbm, send_hbm, recv_hbm, out_hbm,
             sidx0, sidx1, ridx0, ridx1, hrows0, hrows1, mrows0, mrows1,
             agg,
             in0, in1, g0, g1, s0, s1):
        cid = lax.axis_index("c")
        sid = lax.axis_index("s")
        wid = cid * ns + sid

        sidx = (sidx0, sidx1)
        ridx = (ridx0, ridx1)
        hrows = (hrows0, hrows1)
        mrows = (mrows0, mrows1)
        insem = (in0, in1)
        gsem = (g0, g1)
        ssem = (s0, s1)

        zero = jnp.zeros((16,), jnp.float32)

        def zfill(i, carry):
            for c in range(d // 16):
                mrows0[i, pl.ds(c * 16, 16)] = zero
            return carry

        lax.fori_loop(0, _ZROWS, zfill, 0)
        for b in range(n_zero_blocks):
            @pl.when(b % ns == sid)
            def _():
                pltpu.sync_copy(mrows0, agg.at[pl.ds(b * _ZROWS, _ZROWS)])
        plsc.subcore_barrier()

        base0 = wid * e_per_tile

        def issue_loads(base, b):
            pltpu.async_copy(send_hbm.at[pl.ds(base, _K)], sidx[b], insem[b])
            pltpu.async_copy(recv_hbm.at[pl.ds(base, _K)], ridx[b], insem[b])
            pltpu.async_copy(mod_hbm.at[pl.ds(base, _K)], mrows[b], insem[b])

        def wait_loads(base, b):
            pltpu.make_async_copy(send_hbm.at[pl.ds(base, _K)], sidx[b],
                                  insem[b]).wait()
            pltpu.make_async_copy(recv_hbm.at[pl.ds(base, _K)], ridx[b],
                                  insem[b]).wait()
            pltpu.make_async_copy(mod_hbm.at[pl.ds(base, _K)], mrows[b],
                                  insem[b]).wait()

        def issue_gather(b):
            pltpu.async_copy(h_hbm.at[sidx[b]], hrows[b], gsem[b])

        def wait_gather(b):
            pltpu.make_async_copy(h_hbm.at[sidx[b]], hrows[b], gsem[b]).wait()

        def mul(b):
            @plsc.parallel_loop(0, _K, 1, unroll=4)
            def mrow(i):
                for c in range(d // 16):
                    sl = pl.ds(c * 16, 16)
                    mrows[b][i, sl] = mrows[b][i, sl] * hrows[b][i, sl]

        def issue_scatter(b):
            pltpu.async_copy(mrows[b], agg.at[ridx[b]], ssem[b], add=True)

        def wait_scatter(b):
            pltpu.make_async_copy(mrows[b], agg.at[ridx[b]], ssem[b]).wait()

        issue_loads(base0, 0)
        issue_loads(base0 + _K, 1)
        wait_loads(base0, 0)
        issue_gather(0)

        def pair(i, carry):
            c0 = base0 + (2 * i) * _K
            wait_gather(0)
            mul(0)
            issue_scatter(0)
            wait_loads(c0 + _K, 1)
            issue_gather(1)
            wait_scatter(0)
            issue_loads(c0 + 2 * _K, 0)
            wait_gather(1)
            mul(1)
            issue_scatter(1)
            wait_loads(c0 + 2 * _K, 0)
            issue_gather(0)
            wait_scatter(1)
            issue_loads(c0 + 3 * _K, 1)
            return carry

        lax.fori_loop(0, n_pairs_main, pair, 0)

        cA = base0 + (n_chunks - 3) * _K
        wait_gather(0)
        mul(0)
        issue_scatter(0)
        wait_loads(cA + _K, 1)
        issue_gather(1)
        wait_scatter(0)
        issue_loads(cA + 2 * _K, 0)
        wait_gather(1)
        mul(1)
        issue_scatter(1)
        wait_loads(cA + 2 * _K, 0)
        issue_gather(0)
        wait_scatter(1)
        wait_gather(0)
        mul(0)
        issue_scatter(0)
        wait_scatter(0)
        plsc.subcore_barrier()

        for b in range(n_wb_blocks):
            @pl.when(b % ns == sid)
            def _():
                pltpu.sync_copy(agg.at[pl.ds(b * _WROWS, _WROWS)],
                                out_hbm.at[cid, pl.ds(b * _WROWS, _WROWS)])

    return body(h, mod, senders, receivers)


def kernel(node_features, edge_features, radial_embedding, senders, receivers,
           W_up, W_r1, W_r2, W_edge, W_down):
    h = _linear_up(node_features, W_up)
    mod = _edge_mod(radial_embedding, edge_features, W_r1, W_r2, W_edge)
    agg2 = _sc_scatter(h, mod, senders, receivers)
    return _linear_down(agg2, W_down)


# speedup vs baseline: 1.6113x; 1.1369x over previous
import functools

import jax
import jax.numpy as jnp
from jax import lax
from jax.experimental import pallas as pl
from jax.experimental.pallas import tpu as pltpu
from jax.experimental.pallas import tpu_sc as plsc

AVG_NEIGH = 32.0


def _up_body(x_ref, w_ref, o_ref):
    o_ref[...] = jnp.dot(x_ref[...], w_ref[...],
                         preferred_element_type=jnp.float32)


def _mod_body(re_ref, wr1_ref, wr2_ref, wedge_ref, o_ref):
    d_rad = wr1_ref.shape[0]
    d_edge = wedge_ref.shape[0]
    blk = re_ref[...]
    t = jnp.dot(blk[:, :d_rad], wr1_ref[...],
                preferred_element_type=jnp.float32)
    t = t * jax.nn.sigmoid(t)
    rw = jnp.dot(t, wr2_ref[...], preferred_element_type=jnp.float32)
    ew = jnp.dot(blk[:, d_rad:d_rad + d_edge], wedge_ref[...],
                 preferred_element_type=jnp.float32)
    o_ref[...] = rw * ew


def _down_body(a_ref, w_ref, o_ref):
    a = (a_ref[0] + a_ref[1]) * (1.0 / AVG_NEIGH)
    o_ref[...] = jnp.dot(a, w_ref[...], preferred_element_type=jnp.float32)


def _linear_up(node_features, w_up):
    n, d = node_features.shape
    bn = 1000
    return pl.pallas_call(
        _up_body,
        grid=(n // bn,),
        in_specs=[
            pl.BlockSpec((bn, d), lambda i: (i, 0)),
            pl.BlockSpec((d, d), lambda i: (0, 0)),
        ],
        out_specs=pl.BlockSpec((bn, d), lambda i: (i, 0)),
        out_shape=jax.ShapeDtypeStruct((n, d), jnp.float32),
    )(node_features, w_up)


def _edge_mod(re, w_r1, w_r2, w_edge):
    e, d_re = re.shape
    d_rad = w_r1.shape[0]
    hid = w_r1.shape[1]
    d = w_r2.shape[1]
    d_edge = w_edge.shape[0]
    be = 4000
    return pl.pallas_call(
        _mod_body,
        grid=(e // be,),
        in_specs=[
            pl.BlockSpec((be, d_re), lambda i: (i, 0)),
            pl.BlockSpec((d_rad, hid), lambda i: (0, 0)),
            pl.BlockSpec((hid, d), lambda i: (0, 0)),
            pl.BlockSpec((d_edge, d), lambda i: (0, 0)),
        ],
        out_specs=pl.BlockSpec((be, d), lambda i: (i, 0)),
        out_shape=jax.ShapeDtypeStruct((e, d), jnp.float32),
    )(re, w_r1, w_r2, w_edge)


def _linear_down(agg2, w_down):
    _, n, d = agg2.shape
    bn = 1000
    return pl.pallas_call(
        _down_body,
        grid=(n // bn,),
        in_specs=[
            pl.BlockSpec((2, bn, d), lambda i: (0, i, 0)),
            pl.BlockSpec((d, d), lambda i: (0, 0)),
        ],
        out_specs=pl.BlockSpec((bn, d), lambda i: (i, 0)),
        out_shape=jax.ShapeDtypeStruct((n, d), jnp.float32),
    )(agg2, w_down)


_K = 80
_ZROWS = 80
_WROWS = 200


def _sc_scatter(h, mod, senders, receivers):
    n, d = h.shape
    e = senders.shape[0]
    info = plsc.get_sparse_core_info()
    nc, ns = info.num_cores, info.num_subcores
    nw = nc * ns
    e_per_tile = e // nw
    assert e_per_tile * nw == e and e_per_tile % _K == 0
    n_chunks = e_per_tile // _K
    n_zero_blocks = n // _ZROWS
    n_wb_blocks = n // _WROWS
    assert n_zero_blocks * _ZROWS == n and n_wb_blocks * _WROWS == n

    assert n_chunks % 2 == 1 and n_chunks >= 3
    n_pairs_main = (n_chunks - 3) // 2

    mesh = plsc.VectorSubcoreMesh(core_axis_name="c", subcore_axis_name="s",
                                  num_cores=nc, num_subcores=ns)

    @functools.partial(
        pl.kernel,
        mesh=mesh,
        out_type=jax.ShapeDtypeStruct((nc, n, d), jnp.float32),
        scratch_types=[
            pltpu.VMEM((_K,), jnp.int32),
            pltpu.VMEM((_K,), jnp.int32),
            pltpu.VMEM((_K,), jnp.int32),
            pltpu.VMEM((_K,), jnp.int32),
            pltpu.VMEM((_K, d), jnp.float32),
            pltpu.VMEM((_K, d), jnp.float32),
            pltpu.VMEM((_K, d), jnp.float32),
            pltpu.VMEM((_K, d), jnp.float32),
            pltpu.VMEM_SHARED((n, d), jnp.float32),
            pltpu.SemaphoreType.DMA,
            pltpu.SemaphoreType.DMA,
            pltpu.SemaphoreType.DMA,
            pltpu.SemaphoreType.DMA,
            pltpu.SemaphoreType.DMA,
            pltpu.SemaphoreType.DMA,
        ],
    )
    def body(h_hbm, mod_hbm, send_hbm, recv_hbm, out_hbm,
             sidx0, sidx1, ridx0, ridx1, hrows0, hrows1, mrows0, mrows1,
             agg,
             in0, in1, g0, g1, s0, s1):
        cid = lax.axis_index("c")
        sid = lax.axis_index("s")
        wid = cid * ns + sid

        sidx = (sidx0, sidx1)
        ridx = (ridx0, ridx1)
        hrows = (hrows0, hrows1)
        mrows = (mrows0, mrows1)
        insem = (in0, in1)
        gsem = (g0, g1)
        ssem = (s0, s1)

        zero = jnp.zeros((16,), jnp.float32)

        def zfill(i, carry):
            for c in range(d // 16):
                mrows0[i, pl.ds(c * 16, 16)] = zero
            return carry

        lax.fori_loop(0, _ZROWS, zfill, 0)
        for b in range(n_zero_blocks):
            @pl.when(b % ns == sid)
            def _():
                pltpu.sync_copy(mrows0, agg.at[pl.ds(b * _ZROWS, _ZROWS)])
        plsc.subcore_barrier()

        base0 = wid * e_per_tile

        def issue_loads(base, b):
            pltpu.async_copy(send_hbm.at[pl.ds(base, _K)], sidx[b], insem[b])
            pltpu.async_copy(recv_hbm.at[pl.ds(base, _K)], ridx[b], insem[b])
            pltpu.async_copy(mod_hbm.at[pl.ds(base, _K)], mrows[b], insem[b])

        def wait_loads(base, b):
            pltpu.make_async_copy(send_hbm.at[pl.ds(base, _K)], sidx[b],
                                  insem[b]).wait()
            pltpu.make_async_copy(recv_hbm.at[pl.ds(base, _K)], ridx[b],
                                  insem[b]).wait()
            pltpu.make_async_copy(mod_hbm.at[pl.ds(base, _K)], mrows[b],
                                  insem[b]).wait()

        def issue_gather(b):
            pltpu.async_copy(h_hbm.at[sidx[b]], hrows[b], gsem[b])

        def wait_gather(b):
            pltpu.make_async_copy(h_hbm.at[sidx[b]], hrows[b], gsem[b]).wait()

        def mul(b):
            @plsc.parallel_loop(0, _K, 1, unroll=4)
            def mrow(i):
                for c in range(d // 16):
                    sl = pl.ds(c * 16, 16)
                    mrows[b][i, sl] = mrows[b][i, sl] * hrows[b][i, sl]

        def issue_scatter(b):
            pltpu.async_copy(mrows[b], agg.at[ridx[b]], ssem[b], add=True)

        def wait_scatter(b):
            pltpu.make_async_copy(mrows[b], agg.at[ridx[b]], ssem[b]).wait()

        issue_loads(base0, 0)
        issue_loads(base0 + _K, 1)
        wait_loads(base0, 0)
        issue_gather(0)

        def pair(i, carry):
            c0 = base0 + (2 * i) * _K
            wait_gather(0)
            mul(0)
            issue_scatter(0)
            wait_loads(c0 + _K, 1)
            issue_gather(1)
            wait_scatter(0)
            issue_loads(c0 + 2 * _K, 0)
            wait_gather(1)
            mul(1)
            issue_scatter(1)
            wait_loads(c0 + 2 * _K, 0)
            issue_gather(0)
            wait_scatter(1)
            issue_loads(c0 + 3 * _K, 1)
            return carry

        lax.fori_loop(0, n_pairs_main, pair, 0)

        cA = base0 + (n_chunks - 3) * _K
        wait_gather(0)
        mul(0)
        issue_scatter(0)
        wait_loads(cA + _K, 1)
        issue_gather(1)
        wait_scatter(0)
        issue_loads(cA + 2 * _K, 0)
        wait_gather(1)
        mul(1)
        issue_scatter(1)
        wait_loads(cA + 2 * _K, 0)
        issue_gather(0)
        wait_scatter(1)
        wait_gather(0)
        mul(0)
        issue_scatter(0)
        wait_scatter(0)
        plsc.subcore_barrier()

        for b in range(n_wb_blocks):
            @pl.when(b % ns == sid)
            def _():
                pltpu.sync_copy(agg.at[pl.ds(b * _WROWS, _WROWS)],
                                out_hbm.at[cid, pl.ds(b * _WROWS, _WROWS)])

    return body(h, mod, senders, receivers)


def kernel(node_features, edge_features, radial_embedding, senders, receivers,
           W_up, W_r1, W_r2, W_edge, W_down):
    h = _linear_up(node_features, W_up)
    re = jnp.concatenate([radial_embedding, edge_features], axis=1)
    mod = _edge_mod(re, W_r1, W_r2, W_edge)
    agg2 = _sc_scatter(h, mod, senders, receivers)
    return _linear_down(agg2, W_down)
